# Initial kernel scaffold; baseline (speedup 1.0000x reference)
import jax
import jax.numpy as jnp
from jax.experimental import pallas as pl


def kernel(x, edge_index, edge_attr, W1, b1, W2, b2):
    raise NotImplementedError("write your pallas kernel here")


# speedup vs baseline: 11.9558x; 11.9558x over previous
import functools

import jax
import jax.numpy as jnp
from jax import lax
from jax.experimental import pallas as pl
from jax.experimental.pallas import tpu as pltpu
from jax.experimental.pallas import tpu_sc as plsc

NC = 2
NS = 16
LN = 16
K = 80


def _sc_mesh():
  return plsc.VectorSubcoreMesh(core_axis_name="c", subcore_axis_name="s")


def _sc_degree(ch, npad):
  @functools.partial(
      pl.kernel,
      out_type=jax.ShapeDtypeStruct((NC, NS, npad), jnp.float32),
      mesh=_sc_mesh(),
      compiler_params=pltpu.CompilerParams(use_tc_tiling_on_sc=False,
                                           needs_layout_passes=False),
      scratch_types=[
          pltpu.VMEM((ch, K), jnp.int32),
          pltpu.VMEM((ch, K), jnp.float32),
          pltpu.VMEM((npad,), jnp.float32),
      ],
  )
  def deg_kernel(dst_hbm, w_hbm, out_hbm, dst_v, w_v, acc):
    cid = lax.axis_index("c")
    sid = lax.axis_index("s")
    pltpu.sync_copy(dst_hbm.at[cid, sid], dst_v)
    pltpu.sync_copy(w_hbm.at[cid, sid], w_v)

    def zb(i, carry):
      acc[pl.ds(i * LN, LN)] = jnp.zeros((LN,), jnp.float32)
      return carry

    lax.fori_loop(0, npad // LN, zb, 0)

    def body(c, carry):
      for g in range(K // LN):
        idx16 = dst_v[c, pl.ds(g * LN, LN)]
        w16 = w_v[c, pl.ds(g * LN, LN)]
        plsc.addupdate_scatter(acc, [idx16], w16)
      return carry

    lax.fori_loop(0, ch, body, 0)
    pltpu.sync_copy(acc, out_hbm.at[cid, sid])

  return deg_kernel


_BCAST_DNUMS = lax.GatherDimensionNumbers(
    offset_dims=(), collapsed_slice_dims=(0,), start_index_map=(0,))


def _bcast_lane(v, lane):
  idx = jnp.full((LN, 1), lane, dtype=jnp.int32)
  return lax.gather(v, idx, _BCAST_DNUMS, (1,),
                    mode=lax.GatherScatterMode.PROMISE_IN_BOUNDS)


def _sc_aggregate_feat(ch, d, npad):
  rows_t = npad // NS
  zrows = 64
  jcount = d // LN

  @functools.partial(
      pl.kernel,
      out_type=jax.ShapeDtypeStruct((NC, npad, d), jnp.float32),
      mesh=_sc_mesh(),
      compiler_params=pltpu.CompilerParams(use_tc_tiling_on_sc=False),
      scratch_types=[
          pltpu.VMEM((ch, K), jnp.int32),
          pltpu.VMEM((ch, K), jnp.int32),
          pltpu.VMEM((ch, K), jnp.float32),
          pltpu.VMEM((K, d), jnp.float32),
          pltpu.VMEM((zrows, d), jnp.float32),
          pltpu.VMEM_SHARED((npad, d), jnp.float32),
      ],
  )
  def agg_kernel(g_hbm, src_hbm, dst_hbm, w_hbm, out_hbm,
                 src_v, dst_v, w_v, rows, zbuf, acc):
    cid = lax.axis_index("c")
    sid = lax.axis_index("s")
    pltpu.sync_copy(src_hbm.at[sid], src_v)
    pltpu.sync_copy(dst_hbm.at[sid], dst_v)
    pltpu.sync_copy(w_hbm.at[sid], w_v)

    def zb(i, carry):
      for j in range(jcount):
        zbuf[i, pl.ds(j * LN, LN)] = jnp.zeros((LN,), jnp.float32)
      return carry

    lax.fori_loop(0, zrows, zb, 0)

    def zcopy(i, carry):
      pltpu.sync_copy(zbuf, acc.at[pl.ds(sid * rows_t + i * zrows, zrows)])
      return carry

    lax.fori_loop(0, rows_t // zrows, zcopy, 0)
    plsc.subcore_barrier()

    def chunk(c, carry):
      pltpu.sync_copy(g_hbm.at[cid].at[src_v.at[c]], rows)

      def scale(e16, carry2):
        wv = w_v[c, pl.ds(e16 * LN, LN)]
        for lane in range(LN):
          wsplat = _bcast_lane(wv, lane)
          e = e16 * LN + lane
          for j in range(jcount):
            rows[e, pl.ds(j * LN, LN)] = rows[e, pl.ds(j * LN, LN)] * wsplat
        return carry2

      lax.fori_loop(0, K // LN, scale, 0)
      pltpu.sync_copy(rows, acc.at[dst_v.at[c]], add=True)
      return carry

    lax.fori_loop(0, ch, chunk, 0)
    plsc.subcore_barrier()

    def out_copy(i, carry):
      r0 = sid * rows_t + i * zrows
      pltpu.sync_copy(acc.at[pl.ds(r0, zrows)], zbuf)
      pltpu.sync_copy(zbuf, out_hbm.at[cid, pl.ds(r0, zrows)])
      return carry

    lax.fori_loop(0, rows_t // zrows, out_copy, 0)

  return agg_kernel


def _sc_aggregate(ch, d, npad):
  rows_t = npad // NS
  zrows = 64
  jcount = d // LN

  @functools.partial(
      pl.kernel,
      out_type=jax.ShapeDtypeStruct((NC, npad, d), jnp.float32),
      mesh=_sc_mesh(),
      compiler_params=pltpu.CompilerParams(use_tc_tiling_on_sc=False),
      scratch_types=[
          pltpu.VMEM((ch, K), jnp.int32),
          pltpu.VMEM((ch, K), jnp.int32),
          pltpu.VMEM((ch, K), jnp.float32),
          pltpu.VMEM((K, d), jnp.float32),
          pltpu.VMEM((zrows, d), jnp.float32),
          pltpu.VMEM_SHARED((npad, d), jnp.float32),
      ],
  )
  def agg_kernel(g_hbm, src_hbm, dst_hbm, w_hbm, out_hbm,
                 src_v, dst_v, w_v, rows, zbuf, acc):
    cid = lax.axis_index("c")
    sid = lax.axis_index("s")
    pltpu.sync_copy(src_hbm.at[cid, sid], src_v)
    pltpu.sync_copy(dst_hbm.at[cid, sid], dst_v)
    pltpu.sync_copy(w_hbm.at[cid, sid], w_v)

    def zb(i, carry):
      for j in range(jcount):
        zbuf[i, pl.ds(j * LN, LN)] = jnp.zeros((LN,), jnp.float32)
      return carry

    lax.fori_loop(0, zrows, zb, 0)

    def zcopy(i, carry):
      pltpu.sync_copy(zbuf, acc.at[pl.ds(sid * rows_t + i * zrows, zrows)])
      return carry

    lax.fori_loop(0, rows_t // zrows, zcopy, 0)
    plsc.subcore_barrier()

    def chunk(c, carry):
      pltpu.sync_copy(g_hbm.at[src_v.at[c]], rows)

      def scale(e16, carry2):
        wv = w_v[c, pl.ds(e16 * LN, LN)]
        for lane in range(LN):
          wsplat = _bcast_lane(wv, lane)
          e = e16 * LN + lane
          for j in range(jcount):
            rows[e, pl.ds(j * LN, LN)] = rows[e, pl.ds(j * LN, LN)] * wsplat
        return carry2

      lax.fori_loop(0, K // LN, scale, 0)
      pltpu.sync_copy(rows, acc.at[dst_v.at[c]], add=True)
      return carry

    lax.fori_loop(0, ch, chunk, 0)
    plsc.subcore_barrier()

    def out_copy(i, carry):
      r0 = sid * rows_t + i * zrows
      pltpu.sync_copy(acc.at[pl.ds(r0, zrows)], zbuf)
      pltpu.sync_copy(zbuf, out_hbm.at[cid, pl.ds(r0, zrows)])
      return carry

    lax.fori_loop(0, rows_t // zrows, out_copy, 0)

  return agg_kernel


def _tc_layer2(s1, g1, w2p, b2r, dinv_col):
  _, npad, half = s1.shape
  dout = w2p.shape[1]

  def body(s_ref, g_ref, w_ref, b_ref, d_ref, o_ref):
    dinv = d_ref[...]
    z_lo = dinv * (s_ref[0] + g_ref[0])
    z_hi = dinv * (s_ref[1] + g_ref[1])
    a_lo = jnp.where(z_lo > 0, z_lo, jnp.exp(jnp.minimum(z_lo, 0.0)) - 1.0)
    a_hi = jnp.where(z_hi > 0, z_hi, jnp.exp(jnp.minimum(z_hi, 0.0)) - 1.0)
    a = jnp.concatenate([a_lo, a_hi], axis=1)
    h = jnp.dot(a, w_ref[...], preferred_element_type=jnp.float32)
    o_ref[...] = (h + b_ref[...]) * dinv

  return pl.pallas_call(
      body, out_shape=jax.ShapeDtypeStruct((npad, dout), jnp.float32),
  )(s1, g1, w2p, b2r, dinv_col)


def _tc_final(p2, g2, dinv_col, c_out):
  _, npad, dpad = p2.shape

  def body(p_ref, g_ref, d_ref, o_ref):
    z = (d_ref[...] * (p_ref[0] + p_ref[1] + g_ref[...]))[:, :c_out]
    m = jnp.max(z, axis=1, keepdims=True)
    lse = jnp.log(jnp.sum(jnp.exp(z - m), axis=1, keepdims=True)) + m
    o_ref[...] = z - lse

  return pl.pallas_call(
      body, out_shape=jax.ShapeDtypeStruct((npad, c_out), jnp.float32),
  )(p2, g2, dinv_col)


def _tc_layer1(xp, w1, b1r, dinv_col):
  npad, f_in = xp.shape
  hid = w1.shape[1]
  half = hid // 2

  def body(x_ref, w_ref, b_ref, d_ref, g_ref):
    h = jnp.dot(x_ref[...], w_ref[...], preferred_element_type=jnp.float32)
    g = (h + b_ref[...]) * d_ref[...]
    g_ref[0, :, :] = g[:, :half]
    g_ref[1, :, :] = g[:, half:]

  return pl.pallas_call(
      body, out_shape=jax.ShapeDtypeStruct((2, npad, half), jnp.float32),
  )(xp, w1, b1r, dinv_col)


def _tc_dinv(dp):
  nw, npad = dp.shape

  def body(d_ref, o_ref):
    deg = 1.0 + jnp.sum(d_ref[...], axis=0, keepdims=True)
    o_ref[...] = lax.rsqrt(deg)

  return pl.pallas_call(
      body, out_shape=jax.ShapeDtypeStruct((1, npad), jnp.float32),
  )(dp)


def kernel(x, edge_index, edge_attr, W1, b1, W2, b2):
  n, f_in = x.shape
  e = edge_attr.shape[0]
  nw = NC * NS
  ch = e // (nw * K)
  assert e == nw * ch * K
  npad = ((n + nw * 8 - 1) // (nw * 8)) * (nw * 8)

  dst_r = edge_index[1].reshape(NC, NS, ch, K)
  w_r = edge_attr.reshape(NC, NS, ch, K)

  dp = _sc_degree(ch, npad)(dst_r, w_r).reshape(nw, npad)
  dinv_col = _tc_dinv(dp).reshape(npad, 1)
  dinv = dinv_col[:n]

  hid = W1.shape[1]
  chf = e // (NS * K)
  src_f = edge_index[0].reshape(NS, chf, K)
  dst_f = edge_index[1].reshape(NS, chf, K)
  w_f = edge_attr.reshape(NS, chf, K)
  xp = jnp.zeros((npad, f_in), jnp.float32).at[:n].set(x)
  b1r = b1.reshape(1, hid)

  c_out = W2.shape[1]
  d2 = 16
  src_r = edge_index[0].reshape(NC, NS, ch, K)
  w2p = jnp.zeros((hid, d2), jnp.float32).at[:, :c_out].set(W2)
  b2r = jnp.zeros((1, d2), jnp.float32).at[0, :c_out].set(b2)

  g1p = _tc_layer1(xp, W1, b1r, dinv_col)
  s1p = _sc_aggregate_feat(chf, hid // 2, npad)(g1p, src_f, dst_f, w_f)
  g2 = _tc_layer2(s1p, g1p, w2p, b2r, dinv_col)
  p2 = _sc_aggregate(ch, d2, npad)(g2, src_r, dst_r, w_r)
  out = _tc_final(p2, g2, dinv_col, c_out)
  return out[:n]


# speedup vs baseline: 18.8106x; 1.5733x over previous
import functools

import jax
import jax.numpy as jnp
from jax import lax
from jax.experimental import pallas as pl
from jax.experimental.pallas import tpu as pltpu
from jax.experimental.pallas import tpu_sc as plsc

NC = 2
NS = 16
LN = 16
K = 80
G = 5


def _sc_mesh():
  return plsc.VectorSubcoreMesh(core_axis_name="c", subcore_axis_name="s")


def _sc_degree(ch, npad):
  @functools.partial(
      pl.kernel,
      out_type=jax.ShapeDtypeStruct((NC, NS, npad), jnp.float32),
      mesh=_sc_mesh(),
      compiler_params=pltpu.CompilerParams(use_tc_tiling_on_sc=False,
                                           needs_layout_passes=False),
      scratch_types=[
          pltpu.VMEM((ch, K), jnp.int32),
          pltpu.VMEM((ch, K), jnp.float32),
          pltpu.VMEM((npad,), jnp.float32),
      ],
  )
  def deg_kernel(dst_hbm, w_hbm, out_hbm, dst_v, w_v, acc):
    cid = lax.axis_index("c")
    sid = lax.axis_index("s")
    pltpu.sync_copy(dst_hbm.at[cid, sid], dst_v)
    pltpu.sync_copy(w_hbm.at[cid, sid], w_v)

    def zb(i, carry):
      acc[pl.ds(i * LN, LN)] = jnp.zeros((LN,), jnp.float32)
      return carry

    lax.fori_loop(0, npad // LN, zb, 0)

    def body(c, carry):
      for g in range(K // LN):
        idx16 = dst_v[c, pl.ds(g * LN, LN)]
        w16 = w_v[c, pl.ds(g * LN, LN)]
        plsc.addupdate_scatter(acc, [idx16], w16)
      return carry

    lax.fori_loop(0, ch, body, 0)
    pltpu.sync_copy(acc, out_hbm.at[cid, sid])

  return deg_kernel


_BCAST_DNUMS = lax.GatherDimensionNumbers(
    offset_dims=(), collapsed_slice_dims=(0,), start_index_map=(0,))


def _bcast_lane(v, lane):
  idx = jnp.full((LN, 1), lane, dtype=jnp.int32)
  return lax.gather(v, idx, _BCAST_DNUMS, (1,),
                    mode=lax.GatherScatterMode.PROMISE_IN_BOUNDS)


def _batched_chunks(g_src_ref, src_v, dst_v, w_v, rows, acc, sems, ch, d):
  jcount = d // LN

  def batch(t, carry):
    descs = []
    for i in range(G):
      c = t * G + i
      descs.append(pltpu.async_copy(
          g_src_ref.at[src_v.at[c]], rows.at[i], sems[i]))
    sdescs = []
    for i in range(G):
      c = t * G + i
      descs[i].wait()

      def scale(e16, carry2):
        wv = w_v[c, pl.ds(e16 * LN, LN)]
        for lane in range(LN):
          wsplat = _bcast_lane(wv, lane)
          e = e16 * LN + lane
          for j in range(jcount):
            rows[i, e, pl.ds(j * LN, LN)] = (
                rows[i, e, pl.ds(j * LN, LN)] * wsplat)
        return carry2

      lax.fori_loop(0, K // LN, scale, 0)
      sdescs.append(pltpu.async_copy(rows.at[i], acc.at[dst_v.at[c]],
                                     sems[G + i], add=True))
    for i in range(G):
      sdescs[i].wait()
    return carry

  lax.fori_loop(0, ch // G, batch, 0)


def _sc_aggregate_feat(ch, d, npad):
  rows_t = npad // NS
  zrows = 64
  jcount = d // LN

  @functools.partial(
      pl.kernel,
      out_type=jax.ShapeDtypeStruct((NC, npad, d), jnp.float32),
      mesh=_sc_mesh(),
      compiler_params=pltpu.CompilerParams(use_tc_tiling_on_sc=False),
      scratch_types=[
          pltpu.VMEM((ch, K), jnp.int32),
          pltpu.VMEM((ch, K), jnp.int32),
          pltpu.VMEM((ch, K), jnp.float32),
          pltpu.VMEM((G, K, d), jnp.float32),
          pltpu.VMEM((zrows, d), jnp.float32),
          pltpu.VMEM_SHARED((npad, d), jnp.float32),
      ] + [pltpu.SemaphoreType.DMA] * (2 * G),
  )
  def agg_kernel(g_hbm, src_hbm, dst_hbm, w_hbm, out_hbm,
                 src_v, dst_v, w_v, rows, zbuf, acc, *sems):
    cid = lax.axis_index("c")
    sid = lax.axis_index("s")
    pltpu.sync_copy(src_hbm.at[sid], src_v)
    pltpu.sync_copy(dst_hbm.at[sid], dst_v)
    pltpu.sync_copy(w_hbm.at[sid], w_v)

    def zb(i, carry):
      for j in range(jcount):
        zbuf[i, pl.ds(j * LN, LN)] = jnp.zeros((LN,), jnp.float32)
      return carry

    lax.fori_loop(0, zrows, zb, 0)

    def zcopy(i, carry):
      pltpu.sync_copy(zbuf, acc.at[pl.ds(sid * rows_t + i * zrows, zrows)])
      return carry

    lax.fori_loop(0, rows_t // zrows, zcopy, 0)
    plsc.subcore_barrier()

    _batched_chunks(g_hbm.at[cid], src_v, dst_v, w_v, rows, acc, sems, ch, d)
    plsc.subcore_barrier()

    def out_copy(i, carry):
      r0 = sid * rows_t + i * zrows
      pltpu.sync_copy(acc.at[pl.ds(r0, zrows)], zbuf)
      pltpu.sync_copy(zbuf, out_hbm.at[cid, pl.ds(r0, zrows)])
      return carry

    lax.fori_loop(0, rows_t // zrows, out_copy, 0)

  return agg_kernel


def _sc_aggregate(ch, d, npad):
  rows_t = npad // NS
  zrows = 64
  jcount = d // LN

  @functools.partial(
      pl.kernel,
      out_type=jax.ShapeDtypeStruct((NC, npad, d), jnp.float32),
      mesh=_sc_mesh(),
      compiler_params=pltpu.CompilerParams(use_tc_tiling_on_sc=False),
      scratch_types=[
          pltpu.VMEM((ch, K), jnp.int32),
          pltpu.VMEM((ch, K), jnp.int32),
          pltpu.VMEM((ch, K), jnp.float32),
          pltpu.VMEM((G, K, d), jnp.float32),
          pltpu.VMEM((zrows, d), jnp.float32),
          pltpu.VMEM_SHARED((npad, d), jnp.float32),
      ] + [pltpu.SemaphoreType.DMA] * (2 * G),
  )
  def agg_kernel(g_hbm, src_hbm, dst_hbm, w_hbm, out_hbm,
                 src_v, dst_v, w_v, rows, zbuf, acc, *sems):
    cid = lax.axis_index("c")
    sid = lax.axis_index("s")
    pltpu.sync_copy(src_hbm.at[cid, sid], src_v)
    pltpu.sync_copy(dst_hbm.at[cid, sid], dst_v)
    pltpu.sync_copy(w_hbm.at[cid, sid], w_v)

    def zb(i, carry):
      for j in range(jcount):
        zbuf[i, pl.ds(j * LN, LN)] = jnp.zeros((LN,), jnp.float32)
      return carry

    lax.fori_loop(0, zrows, zb, 0)

    def zcopy(i, carry):
      pltpu.sync_copy(zbuf, acc.at[pl.ds(sid * rows_t + i * zrows, zrows)])
      return carry

    lax.fori_loop(0, rows_t // zrows, zcopy, 0)
    plsc.subcore_barrier()

    _batched_chunks(g_hbm, src_v, dst_v, w_v, rows, acc, sems, ch, d)
    plsc.subcore_barrier()

    def out_copy(i, carry):
      r0 = sid * rows_t + i * zrows
      pltpu.sync_copy(acc.at[pl.ds(r0, zrows)], zbuf)
      pltpu.sync_copy(zbuf, out_hbm.at[cid, pl.ds(r0, zrows)])
      return carry

    lax.fori_loop(0, rows_t // zrows, out_copy, 0)

  return agg_kernel


def _tc_layer2(s1, g1, w2p, b2r, dinv_col):
  _, npad, half = s1.shape
  dout = w2p.shape[1]

  def body(s_ref, g_ref, w_ref, b_ref, d_ref, o_ref):
    dinv = d_ref[...]
    z_lo = dinv * (s_ref[0] + g_ref[0])
    z_hi = dinv * (s_ref[1] + g_ref[1])
    a_lo = jnp.where(z_lo > 0, z_lo, jnp.exp(jnp.minimum(z_lo, 0.0)) - 1.0)
    a_hi = jnp.where(z_hi > 0, z_hi, jnp.exp(jnp.minimum(z_hi, 0.0)) - 1.0)
    a = jnp.concatenate([a_lo, a_hi], axis=1)
    h = jnp.dot(a, w_ref[...], preferred_element_type=jnp.float32)
    o_ref[...] = (h + b_ref[...]) * dinv

  return pl.pallas_call(
      body, out_shape=jax.ShapeDtypeStruct((npad, dout), jnp.float32),
  )(s1, g1, w2p, b2r, dinv_col)


def _tc_final(p2, g2, dinv_col, c_out):
  _, npad, dpad = p2.shape

  def body(p_ref, g_ref, d_ref, o_ref):
    z = (d_ref[...] * (p_ref[0] + p_ref[1] + g_ref[...]))[:, :c_out]
    m = jnp.max(z, axis=1, keepdims=True)
    lse = jnp.log(jnp.sum(jnp.exp(z - m), axis=1, keepdims=True)) + m
    o_ref[...] = z - lse

  return pl.pallas_call(
      body, out_shape=jax.ShapeDtypeStruct((npad, c_out), jnp.float32),
  )(p2, g2, dinv_col)


def _tc_layer1(xp, w1, b1r, dinv_col):
  npad, f_in = xp.shape
  hid = w1.shape[1]
  half = hid // 2

  def body(x_ref, w_ref, b_ref, d_ref, g_ref):
    h = jnp.dot(x_ref[...], w_ref[...], preferred_element_type=jnp.float32)
    g = (h + b_ref[...]) * d_ref[...]
    g_ref[0, :, :] = g[:, :half]
    g_ref[1, :, :] = g[:, half:]

  return pl.pallas_call(
      body, out_shape=jax.ShapeDtypeStruct((2, npad, half), jnp.float32),
  )(xp, w1, b1r, dinv_col)


def _tc_dinv(dp):
  nw, npad = dp.shape

  def body(d_ref, o_ref):
    deg = 1.0 + jnp.sum(d_ref[...], axis=0, keepdims=True)
    o_ref[...] = lax.rsqrt(deg)

  return pl.pallas_call(
      body, out_shape=jax.ShapeDtypeStruct((1, npad), jnp.float32),
  )(dp)


def kernel(x, edge_index, edge_attr, W1, b1, W2, b2):
  n, f_in = x.shape
  e = edge_attr.shape[0]
  nw = NC * NS
  ch = e // (nw * K)
  assert e == nw * ch * K
  npad = ((n + nw * 8 - 1) // (nw * 8)) * (nw * 8)

  dst_r = edge_index[1].reshape(NC, NS, ch, K)
  w_r = edge_attr.reshape(NC, NS, ch, K)

  dp = _sc_degree(ch, npad)(dst_r, w_r).reshape(nw, npad)
  dinv_col = _tc_dinv(dp).reshape(npad, 1)
  dinv = dinv_col[:n]

  hid = W1.shape[1]
  chf = e // (NS * K)
  src_f = edge_index[0].reshape(NS, chf, K)
  dst_f = edge_index[1].reshape(NS, chf, K)
  w_f = edge_attr.reshape(NS, chf, K)
  xp = jnp.zeros((npad, f_in), jnp.float32).at[:n].set(x)
  b1r = b1.reshape(1, hid)

  c_out = W2.shape[1]
  d2 = 16
  src_r = edge_index[0].reshape(NC, NS, ch, K)
  w2p = jnp.zeros((hid, d2), jnp.float32).at[:, :c_out].set(W2)
  b2r = jnp.zeros((1, d2), jnp.float32).at[0, :c_out].set(b2)

  g1p = _tc_layer1(xp, W1, b1r, dinv_col)
  s1p = _sc_aggregate_feat(chf, hid // 2, npad)(g1p, src_f, dst_f, w_f)
  g2 = _tc_layer2(s1p, g1p, w2p, b2r, dinv_col)
  p2 = _sc_aggregate(ch, d2, npad)(g2, src_r, dst_r, w_r)
  out = _tc_final(p2, g2, dinv_col, c_out)
  return out[:n]
